# baseline (device time: 89706 ns/iter reference)
import jax
import jax.numpy as jnp
from jax import lax
from jax.experimental import pallas as pl
from jax.experimental.pallas import tpu as pltpu

N_DEV = 4
NSUB = 2
COMM_DTYPE = jnp.bfloat16


def kernel(x, w_mat, scale_x, scale_w):
    m, _k = x.shape
    _, n = w_mat.shape
    m_per = m // N_DEV
    half = n // 2
    sub = half // NSUB

    x = x.astype(jnp.float8_e4m3fn)
    w_mat = w_mat.astype(jnp.float8_e4m3fn)

    def body(x_ref, w_ref, sx_ref, sw_ref, out_ref,
             stage_r, stage_l,
             send_sems_r, recv_sems_r, send_sems_l, recv_sems_l):
        my = lax.axis_index("i")
        left = lax.rem(my + N_DEV - 1, N_DEV)
        right = lax.rem(my + 1, N_DEV)

        barrier_sem = pltpu.get_barrier_semaphore()
        for nbr in (left, right):
            pl.semaphore_signal(
                barrier_sem, inc=1,
                device_id=(nbr,), device_id_type=pl.DeviceIdType.MESH,
            )
        pl.semaphore_wait(barrier_sem, 2)

        def partial(c, col0):
            xc = x_ref[pl.ds(c * m_per, m_per), :]
            return jnp.dot(xc, w_ref[:, col0:col0 + sub],
                           preferred_element_type=jnp.float32)

        scale = sx_ref[0] * sw_ref[0]

        dirs = [
            (stage_r, send_sems_r, recv_sems_r, right, 0,
             lambda h: lax.rem(my + 2 * N_DEV - 2 - h, N_DEV)),
            (stage_l, send_sems_l, recv_sems_l, left, half,
             lambda h: lax.rem(my + 2 + h, N_DEV)),
        ]

        def make_rdma(h, d, b):
            stage, ssems, rsems, nbr, col0, _ = dirs[d]
            return pltpu.make_async_remote_copy(
                src_ref=stage.at[h, :, pl.ds(b * sub, sub)],
                dst_ref=stage.at[h + 1, :, pl.ds(b * sub, sub)],
                send_sem=ssems.at[h, b],
                recv_sem=rsems.at[h, b],
                device_id=(nbr,), device_id_type=pl.DeviceIdType.MESH,
            )

        sends = {}

        for b in range(NSUB):
            for d in range(2):
                stage, _, _, nbr, col0, chunk_at = dirs[d]
                c0 = chunk_at(-1)
                stage[0, :, pl.ds(b * sub, sub)] = (
                    partial(c0, col0 + b * sub).astype(COMM_DTYPE))
                r = make_rdma(0, d, b)
                r.start()
                sends[(0, d, b)] = r

        for h in range(N_DEV - 1):
            last = h == N_DEV - 2
            for b in range(NSUB):
                for d in range(2):
                    stage, _, _, nbr, col0, chunk_at = dirs[d]
                    c = chunk_at(h)
                    p = partial(c, col0 + b * sub)
                    sends[(h, d, b)].wait_recv()
                    t = stage[h + 1, :, pl.ds(b * sub, sub)].astype(
                        jnp.float32) + p
                    if not last:
                        stage[h + 1, :, pl.ds(b * sub, sub)] = (
                            t.astype(COMM_DTYPE))
                        r = make_rdma(h + 1, d, b)
                        r.start()
                        sends[(h + 1, d, b)] = r
                    else:
                        out_ref[:, pl.ds(col0 + b * sub, sub)] = t * scale

        for r in sends.values():
            r.wait_send()

    return pl.pallas_call(
        body,
        out_shape=jax.ShapeDtypeStruct((m_per, n), jnp.float32),
        in_specs=[
            pl.BlockSpec(memory_space=pltpu.VMEM),
            pl.BlockSpec(memory_space=pltpu.VMEM),
            pl.BlockSpec(memory_space=pltpu.SMEM),
            pl.BlockSpec(memory_space=pltpu.SMEM),
        ],
        out_specs=pl.BlockSpec(memory_space=pltpu.VMEM),
        scratch_shapes=[
            pltpu.VMEM((N_DEV, m_per, half), COMM_DTYPE),
            pltpu.VMEM((N_DEV, m_per, half), COMM_DTYPE),
            pltpu.SemaphoreType.DMA((N_DEV - 1, NSUB)),
            pltpu.SemaphoreType.DMA((N_DEV - 1, NSUB)),
            pltpu.SemaphoreType.DMA((N_DEV - 1, NSUB)),
            pltpu.SemaphoreType.DMA((N_DEV - 1, NSUB)),
        ],
        compiler_params=pltpu.CompilerParams(collective_id=0),
    )(x, w_mat, scale_x, scale_w)


# device time: 88292 ns/iter; 1.0160x vs baseline; 1.0160x over previous
import jax
import jax.numpy as jnp
from jax import lax
from jax.experimental import pallas as pl
from jax.experimental.pallas import tpu as pltpu

N_DEV = 4
NSUB = 2
COMM_DTYPE = jnp.bfloat16
GEMM_DTYPE = jnp.float8_e4m3fn


def kernel(x, w_mat, scale_x, scale_w):
    m, k = x.shape
    _, n = w_mat.shape
    m_per = m // N_DEV
    half = n // 2
    sub = half // NSUB

    def body(x_ref, w_ref, sx_ref, sw_ref, out_ref,
             stage_r, stage_l, x8, w8, xbuf, wbuf,
             xsems, wsems,
             send_sems_r, recv_sems_r, send_sems_l, recv_sems_l):
        my = lax.axis_index("i")
        left = lax.rem(my + N_DEV - 1, N_DEV)
        right = lax.rem(my + 1, N_DEV)

        c_first = lax.rem(my + N_DEV - 1, N_DEV)
        c_second = lax.rem(my + 1, N_DEV)
        c_third = lax.rem(my + 2, N_DEV)

        def copy_x(c, slot, sem_idx):
            return pltpu.make_async_copy(
                x_ref.at[pl.ds(c * m_per, m_per), :],
                xbuf.at[slot], xsems.at[sem_idx])

        cp0 = copy_x(c_first, 0, 0)
        cp1 = copy_x(c_second, 1, 1)
        cp0.start()
        cp1.start()
        wblocks = [d * NSUB + b for b in range(NSUB) for d in range(2)]
        wcps = []
        for j, wb in enumerate(wblocks):
            cp = pltpu.make_async_copy(
                w_ref.at[:, pl.ds(wb * sub, sub)],
                wbuf.at[:, pl.ds(wb * sub, sub)], wsems.at[j])
            cp.start()
            wcps.append(cp)

        barrier_sem = pltpu.get_barrier_semaphore()
        for nbr in (left, right):
            pl.semaphore_signal(
                barrier_sem, inc=1,
                device_id=(nbr,), device_id_type=pl.DeviceIdType.MESH,
            )
        pl.semaphore_wait(barrier_sem, 2)

        def land_x(cp, c, slot):
            cp.wait()
            x8[pl.ds(c * m_per, m_per), :] = xbuf[slot].astype(GEMM_DTYPE)

        def land_w(j):
            wcps[j].wait()
            wb = wblocks[j]
            sl = pl.ds(wb * sub, sub)
            w8[:, sl] = wbuf[:, sl].astype(GEMM_DTYPE)

        def partial(c, col0):
            xc = x8[pl.ds(c * m_per, m_per), :]
            return jnp.dot(xc, w8[:, col0:col0 + sub],
                           preferred_element_type=jnp.float32)

        scale = sx_ref[0] * sw_ref[0]

        dirs = [
            (stage_r, send_sems_r, recv_sems_r, right, 0,
             lambda h: lax.rem(my + 2 * N_DEV - 2 - h, N_DEV)),
            (stage_l, send_sems_l, recv_sems_l, left, half,
             lambda h: lax.rem(my + 2 + h, N_DEV)),
        ]

        def make_rdma(h, d, b):
            stage, ssems, rsems, nbr, col0, _ = dirs[d]
            return pltpu.make_async_remote_copy(
                src_ref=stage.at[h, :, pl.ds(b * sub, sub)],
                dst_ref=stage.at[h + 1, :, pl.ds(b * sub, sub)],
                send_sem=ssems.at[h, b],
                recv_sem=rsems.at[h, b],
                device_id=(nbr,), device_id_type=pl.DeviceIdType.MESH,
            )

        sends = {}
        cp2 = cp3 = None

        for b in range(NSUB):
            for d in range(2):
                stage, _, _, nbr, col0, chunk_at = dirs[d]
                if b == 0:
                    if d == 0:
                        land_x(cp0, c_first, 0)
                        cp2 = copy_x(c_third, 0, 2)
                        cp2.start()
                    else:
                        land_x(cp1, c_second, 1)
                        cp3 = copy_x(my, 1, 3)
                        cp3.start()
                land_w(b * 2 + d)
                c0 = chunk_at(-1)
                stage[0, :, pl.ds(b * sub, sub)] = (
                    partial(c0, col0 + b * sub).astype(COMM_DTYPE))
                r = make_rdma(0, d, b)
                r.start()
                sends[(0, d, b)] = r

        for h in range(N_DEV - 1):
            if h == 0:
                land_x(cp2, c_third, 0)
            if h == N_DEV - 2:
                land_x(cp3, my, 1)
            last = h == N_DEV - 2
            for b in range(NSUB):
                for d in range(2):
                    stage, _, _, nbr, col0, chunk_at = dirs[d]
                    c = chunk_at(h)
                    p = partial(c, col0 + b * sub)
                    sends[(h, d, b)].wait_recv()
                    t = stage[h + 1, :, pl.ds(b * sub, sub)].astype(
                        jnp.float32) + p
                    if not last:
                        stage[h + 1, :, pl.ds(b * sub, sub)] = (
                            t.astype(COMM_DTYPE))
                        r = make_rdma(h + 1, d, b)
                        r.start()
                        sends[(h + 1, d, b)] = r
                    else:
                        out_ref[:, pl.ds(col0 + b * sub, sub)] = t * scale

        for r in sends.values():
            r.wait_send()

    return pl.pallas_call(
        body,
        out_shape=jax.ShapeDtypeStruct((m_per, n), jnp.float32),
        in_specs=[
            pl.BlockSpec(memory_space=pl.ANY),
            pl.BlockSpec(memory_space=pl.ANY),
            pl.BlockSpec(memory_space=pltpu.SMEM),
            pl.BlockSpec(memory_space=pltpu.SMEM),
        ],
        out_specs=pl.BlockSpec(memory_space=pltpu.VMEM),
        scratch_shapes=[
            pltpu.VMEM((N_DEV, m_per, half), COMM_DTYPE),
            pltpu.VMEM((N_DEV, m_per, half), COMM_DTYPE),
            pltpu.VMEM((m, k), GEMM_DTYPE),
            pltpu.VMEM((k, n), GEMM_DTYPE),
            pltpu.VMEM((2, m_per, k), jnp.float32),
            pltpu.VMEM((k, n), jnp.float32),
            pltpu.SemaphoreType.DMA((N_DEV,)),
            pltpu.SemaphoreType.DMA((2 * NSUB,)),
            pltpu.SemaphoreType.DMA((N_DEV - 1, NSUB)),
            pltpu.SemaphoreType.DMA((N_DEV - 1, NSUB)),
            pltpu.SemaphoreType.DMA((N_DEV - 1, NSUB)),
            pltpu.SemaphoreType.DMA((N_DEV - 1, NSUB)),
        ],
        compiler_params=pltpu.CompilerParams(
            collective_id=0,
            vmem_limit_bytes=52 * 1024 * 1024,
        ),
    )(x, w_mat, scale_x, scale_w)
